# Initial kernel scaffold; baseline (speedup 1.0000x reference)
import jax
import jax.numpy as jnp
from jax.experimental import pallas as pl


def kernel(xyz, points, params):
    raise NotImplementedError("write your pallas kernel here")


# speedup vs baseline: 2.7890x; 2.7890x over previous
import functools
import jax
import jax.numpy as jnp
from jax import lax
from jax.experimental import pallas as pl
from jax.experimental.pallas import tpu as pltpu
from jax.experimental.pallas import tpu_sc as plsc

_NPOINT = 512
_NSAMPLE = 32
_B, _D, _N = 4, 64, 2048
_ROWS = _B * _NPOINT * _NSAMPLE
_TBL_C = 128
_QB = 64
_RB = _QB * _NSAMPLE
_NBLK = _ROWS // _RB
_EPS = 1e-5


def _sqd_sel(a, xb):
    nb = jnp.sum(xb * xb, axis=0, keepdims=True)
    d = -2.0 * lax.dot_general(a, xb, (((0,), (0,)), ((), ())),
                               preferred_element_type=jnp.float32)
    d = d + nb
    return d


def _density_body(xyz_ref, xq_ref, xqr_ref, out_ref):
    xb = xyz_ref[0]
    a = xq_ref[0]
    ar = xqr_ref[0]
    d0 = _sqd_sel(a, xb)
    m0 = jnp.mean(jnp.exp(d0 * -0.5), axis=-1, keepdims=True)
    na = jnp.sum(ar * ar, axis=1, keepdims=True)
    out_ref[0] = 2.5 * jnp.exp(na * 0.5) / m0


def _density(xyz, xyz_rows):
    out = pl.pallas_call(
        _density_body,
        grid=(_B, _N // 512),
        in_specs=[pl.BlockSpec((1, 3, _N), lambda b, q: (b, 0, 0)),
                  pl.BlockSpec((1, 3, 512), lambda b, q: (b, 0, q)),
                  pl.BlockSpec((1, 512, 3), lambda b, q: (b, q, 0))],
        out_specs=pl.BlockSpec((1, 512, 1), lambda b, q: (b, q, 0)),
        out_shape=jax.ShapeDtypeStruct((_B, _N, 1), jnp.float32),
    )(xyz, xyz, xyz_rows)
    return out.reshape(_B, _N)


def _fps_body(xyz_ref, idx_ref, nxyz_ref):
    xcs = [[xyz_ref[b, k:k + 1, :] for k in range(3)] for b in range(_B)]
    lane_n = lax.broadcasted_iota(jnp.int32, (1, _N), 1)
    lane_p = lax.broadcasted_iota(jnp.int32, (1, _NPOINT), 1)

    def step(i, carry):
        dists, fars, cents, nxs = carry
        mask_i = (lane_p == i).astype(jnp.int32)
        mask_f = mask_i.astype(jnp.float32)
        out = [[], [], [], []]
        for b in range(_B):
            far = fars[b]
            cent = cents[b] + mask_i * far
            onehot = (lane_n == far).astype(jnp.float32)
            cs = [jnp.sum(xcs[b][k] * onehot) for k in range(3)]
            nx = tuple(nxs[b][k] + mask_f * cs[k] for k in range(3))
            d = ((xcs[b][0] - cs[0]) ** 2 + (xcs[b][1] - cs[1]) ** 2
                 + (xcs[b][2] - cs[2]) ** 2)
            dist = jnp.minimum(dists[b], d)
            far2 = jnp.argmax(dist).astype(jnp.int32)
            out[0].append(dist)
            out[1].append(far2)
            out[2].append(cent)
            out[3].append(nx)
        return tuple(tuple(x) for x in out)

    carry0 = (
        tuple(jnp.full((1, _N), 1e10, jnp.float32) for _ in range(_B)),
        tuple(jnp.int32(0) for _ in range(_B)),
        tuple(jnp.zeros((1, _NPOINT), jnp.int32) for _ in range(_B)),
        tuple(tuple(jnp.zeros((1, _NPOINT), jnp.float32) for _ in range(3))
              for _ in range(_B)),
    )
    _, _, cents, nxs = lax.fori_loop(0, _NPOINT, step, carry0)
    for b in range(_B):
        idx_ref[b] = cents[b]
        for k in range(3):
            nxyz_ref[b, k:k + 1, :] = nxs[b][k]


def _fps(xyz):
    idx, nxyz = pl.pallas_call(
        _fps_body,
        grid=(1,),
        in_specs=[pl.BlockSpec((_B, 3, _N), lambda i: (0, 0, 0))],
        out_specs=[
            pl.BlockSpec((_B, 1, _NPOINT), lambda i: (0, 0, 0)),
            pl.BlockSpec((_B, 3, _NPOINT), lambda i: (0, 0, 0)),
        ],
        out_shape=[
            jax.ShapeDtypeStruct((_B, 1, _NPOINT), jnp.int32),
            jax.ShapeDtypeStruct((_B, 3, _NPOINT), jnp.float32),
        ],
    )(xyz)
    return idx.reshape(_B, _NPOINT), nxyz


def _knn_body(nxyz_ref, xyz_ref, invd_ref, idx_ref, gdens_ref):
    a = nxyz_ref[0]
    xb = xyz_ref[0]
    invd = invd_ref[0]
    d = _sqd_sel(a, xb)
    col = lax.broadcasted_iota(jnp.int32, (_NPOINT, _N), 1)
    for k in range(_NSAMPLE):
        am = jnp.argmin(d, axis=-1).astype(jnp.int32)
        idx_ref[0, k, :] = am
        maskf = (col == am[:, None]).astype(jnp.float32)
        gdens_ref[0, k, :] = jnp.sum(maskf * invd, axis=-1)
        d = d + maskf * 1e30


def _knn(new_xyz_cn, xyz, inv_d):
    return pl.pallas_call(
        _knn_body,
        grid=(_B,),
        in_specs=[
            pl.BlockSpec((1, 3, _NPOINT), lambda b: (b, 0, 0)),
            pl.BlockSpec((1, 3, _N), lambda b: (b, 0, 0)),
            pl.BlockSpec((1, 1, _N), lambda b: (b, 0, 0)),
        ],
        out_specs=[
            pl.BlockSpec((1, _NSAMPLE, _NPOINT), lambda b: (b, 0, 0)),
            pl.BlockSpec((1, _NSAMPLE, _NPOINT), lambda b: (b, 0, 0)),
        ],
        out_shape=[
            jax.ShapeDtypeStruct((_B, _NSAMPLE, _NPOINT), jnp.int32),
            jax.ShapeDtypeStruct((_B, _NSAMPLE, _NPOINT), jnp.float32),
        ],
    )(new_xyz_cn, xyz, inv_d.reshape(_B, 1, _N))


def _dna_body(gd_ref, w0_ref, b0_ref, mom_ref):
    @pl.when(pl.program_id(0) == 0)
    def _():
        for c in range(16):
            mom_ref[0, c] = 0.0
            mom_ref[1, c] = 0.0

    gd = gd_ref[0]
    mx = jnp.max(gd, axis=0, keepdims=True)
    ds = gd / mx
    for c in range(16):
        h = ds * w0_ref[c] + b0_ref[c]
        mom_ref[0, c] += jnp.sum(h)
        mom_ref[1, c] += jnp.sum(h * h)


def _dnb_body(gd_ref, w0_ref, b0_ref, sc_ref, w1_ref, b1_ref,
              z_ref, mom_ref):
    @pl.when(pl.program_id(0) == 0)
    def _():
        mom_ref[0, 0] = 0.0
        mom_ref[1, 0] = 0.0

    gd = gd_ref[0]
    mx = jnp.max(gd, axis=0, keepdims=True)
    ds = gd / mx
    z = jnp.zeros((_NSAMPLE, _NPOINT), jnp.float32)
    for c in range(16):
        h = ds * w0_ref[c] + b0_ref[c]
        a = jnp.maximum(h * sc_ref[0, c] + sc_ref[1, c], 0.0)
        z = z + a * w1_ref[c]
    z = z + b1_ref[0]
    z_ref[0] = z
    mom_ref[0, 0] += jnp.sum(z)
    mom_ref[1, 0] += jnp.sum(z * z)


def _dnc_body(z_ref, sc_ref, out_ref):
    dsf = jnp.maximum(z_ref[0] * sc_ref[0, 0] + sc_ref[1, 0], 0.0)
    out_ref[0] = jnp.transpose(dsf)


def _density_scale(gdens, p):
    w0 = p["dn0_w"].reshape(16)
    b0 = p["dn0_b"]
    smem = functools.partial(pl.BlockSpec, memory_space=pltpu.SMEM)
    gd_spec = pl.BlockSpec((1, _NSAMPLE, _NPOINT), lambda b: (b, 0, 0))

    m1 = pl.pallas_call(
        _dna_body,
        grid=(_B,),
        in_specs=[gd_spec, smem((16,), lambda b: (0,)),
                  smem((16,), lambda b: (0,))],
        out_specs=smem((2, 16), lambda b: (0, 0)),
        out_shape=jax.ShapeDtypeStruct((2, 16), jnp.float32),
    )(gdens, w0, b0)

    cnt = jnp.float32(_ROWS)
    sa, sb = _scale_shift(m1[0], m1[1], cnt, p["dn0_g"], p["dn0_be"])
    sc1 = jnp.stack([sa, sb])

    z, m2 = pl.pallas_call(
        _dnb_body,
        grid=(_B,),
        in_specs=[gd_spec, smem((16,), lambda b: (0,)),
                  smem((16,), lambda b: (0,)), smem((2, 16), lambda b: (0, 0)),
                  smem((16,), lambda b: (0,)), smem((1,), lambda b: (0,))],
        out_specs=[gd_spec, smem((2, 1), lambda b: (0, 0))],
        out_shape=[jax.ShapeDtypeStruct((_B, _NSAMPLE, _NPOINT), jnp.float32),
                   jax.ShapeDtypeStruct((2, 1), jnp.float32)],
    )(gdens, w0, b0, sc1, p["dn1_w"].reshape(16), p["dn1_b"])

    za, zb = _scale_shift(m2[0], m2[1], cnt, p["dn1_g"], p["dn1_be"])
    sc2 = jnp.stack([za, zb])

    dsf = pl.pallas_call(
        _dnc_body,
        grid=(_B,),
        in_specs=[gd_spec, smem((2, 1), lambda b: (0, 0))],
        out_specs=pl.BlockSpec((1, _NPOINT, _NSAMPLE), lambda b: (b, 0, 0)),
        out_shape=jax.ShapeDtypeStruct((_B, _NPOINT, _NSAMPLE), jnp.float32),
    )(z, sc2)
    return dsf.reshape(_ROWS, 1)


def _sc_gather(table, flat_idx):
    info = plsc.get_sparse_core_info()
    nw = info.num_cores * info.num_subcores
    per_w = _ROWS // nw
    chunk = 128
    nchunk = per_w // chunk
    mesh = plsc.VectorSubcoreMesh(core_axis_name="c", subcore_axis_name="s")

    @functools.partial(
        pl.kernel, mesh=mesh,
        out_type=jax.ShapeDtypeStruct((_ROWS, _TBL_C), jnp.float32),
        scratch_types=[
            pltpu.VMEM((chunk,), jnp.int32),
            pltpu.VMEM((chunk, _TBL_C), jnp.float32),
            pltpu.SemaphoreType.DMA,
        ],
    )
    def gk(table_hbm, idx_hbm, out_hbm, idx_v, rows_v, sem):
        wid = lax.axis_index("s") * info.num_cores + lax.axis_index("c")
        base = wid * per_w
        for c in range(nchunk):
            off = base + c * chunk
            pltpu.sync_copy(idx_hbm.at[pl.ds(off, chunk)], idx_v)
            pltpu.async_copy(table_hbm.at[idx_v], rows_v, sem).wait()
            pltpu.sync_copy(rows_v, out_hbm.at[pl.ds(off, chunk)])

    return gk(table, flat_idx)


def _acc_moments(mom_ref, cols, off, y):
    s = jnp.sum(y, axis=0, keepdims=True)
    sq = jnp.sum(y * y, axis=0, keepdims=True)
    mom_ref[0:1, off:off + cols] += s
    mom_ref[1:2, off:off + cols] += sq


def _k1_body(g_ref, nx_ref, w1_ref, b1_ref, wn0_ref, bn0_ref,
             y1_ref, wy1_ref, mom_ref):
    @pl.when(pl.program_id(0) == 0)
    def _():
        mom_ref[...] = jnp.zeros((8, 128), jnp.float32)

    g = g_ref[...]
    nx = nx_ref[...]
    xg = g[:, 64:67] - nx
    x67 = jnp.concatenate([xg, g[:, 0:64]], axis=1)
    y1 = lax.dot_general(x67, w1_ref[...], (((1,), (1,)), ((), ())),
                         preferred_element_type=jnp.float32, precision=lax.Precision.HIGHEST) + b1_ref[...]
    y1_ref[...] = y1
    _acc_moments(mom_ref, 64, 0, y1)

    wy = lax.dot_general(xg, wn0_ref[...], (((1,), (1,)), ((), ())),
                         preferred_element_type=jnp.float32, precision=lax.Precision.HIGHEST) + bn0_ref[...]
    wy1_ref[...] = wy
    _acc_moments(mom_ref, 8, 64, wy)


def _k2_body(y1_ref, wy1_ref, sc_ref, w2_ref, b2_ref, wn1_ref,
             bn1_ref, y2_ref, wy2_ref, mom_ref):
    @pl.when(pl.program_id(0) == 0)
    def _():
        mom_ref[...] = jnp.zeros((8, 128), jnp.float32)

    sc = sc_ref[...]
    a1 = jnp.maximum(y1_ref[...] * sc[0:1, 0:64] + sc[1:2, 0:64], 0.0)
    y2 = lax.dot_general(a1, w2_ref[...], (((1,), (1,)), ((), ())),
                         preferred_element_type=jnp.float32, precision=lax.Precision.HIGHEST) + b2_ref[...]
    y2_ref[...] = y2
    _acc_moments(mom_ref, 64, 0, y2)

    wa = jnp.maximum(wy1_ref[...] * sc[0:1, 64:72] + sc[1:2, 64:72], 0.0)
    wy = lax.dot_general(wa, wn1_ref[...], (((1,), (1,)), ((), ())),
                         preferred_element_type=jnp.float32, precision=lax.Precision.HIGHEST) + bn1_ref[...]
    wy2_ref[...] = wy
    _acc_moments(mom_ref, 8, 64, wy)


def _k3_body(y2_ref, wy2_ref, sc_ref, w3_ref, b3_ref, wn2_ref, bn2_ref,
             y3_ref, wy3_ref, mom_ref):
    @pl.when(pl.program_id(0) == 0)
    def _():
        mom_ref[...] = jnp.zeros((8, 128), jnp.float32)

    sc = sc_ref[...]
    a2 = jnp.maximum(y2_ref[...] * sc[0:1, 0:64] + sc[1:2, 0:64], 0.0)
    y3 = lax.dot_general(a2, w3_ref[...], (((1,), (1,)), ((), ())),
                         preferred_element_type=jnp.float32, precision=lax.Precision.HIGHEST) + b3_ref[...]
    y3_ref[...] = y3
    _acc_moments(mom_ref, 128, 0, y3)

    wa = jnp.maximum(wy2_ref[...] * sc[0:1, 64:72] + sc[1:2, 64:72], 0.0)
    wy = lax.dot_general(wa, wn2_ref[...], (((1,), (1,)), ((), ())),
                         preferred_element_type=jnp.float32, precision=lax.Precision.HIGHEST) + bn2_ref[...]
    wy3_ref[...] = wy
    s = jnp.sum(wy, axis=0, keepdims=True)
    sq = jnp.sum(wy * wy, axis=0, keepdims=True)
    mom_ref[2:3, 0:16] += s
    mom_ref[3:4, 0:16] += sq


def _k4_body(y3_ref, dsf_ref, wy3_ref, sc_ref, lin_ref, linb_ref,
             y4_ref, mom_ref):
    @pl.when(pl.program_id(0) == 0)
    def _():
        mom_ref[...] = jnp.zeros((8, 128), jnp.float32)

    sc = sc_ref[...]
    a3 = jnp.maximum(y3_ref[...] * sc[0:1, :] + sc[1:2, :], 0.0)
    dsf = dsf_ref[...]
    dshi = dsf.astype(jnp.bfloat16).astype(jnp.float32)
    ones_r = jnp.ones((128, 1), jnp.float32)
    dn1 = (((1,), (1,)), ((), ()))
    ds128 = (lax.dot_general(dshi, ones_r, dn1,
                             preferred_element_type=jnp.float32)
             + lax.dot_general(dsf - dshi, ones_r, dn1,
                               preferred_element_type=jnp.float32))
    x = a3 * ds128
    w = jnp.maximum(wy3_ref[...] * sc[2:3, 16:32] + sc[3:4, 16:32], 0.0)
    xr = x.reshape(_QB, _NSAMPLE, 128)
    wr = w.reshape(_QB, _NSAMPLE, 16)
    o = lax.dot_general(xr, wr, (((1,), (1,)), ((0,), (0,))),
                        preferred_element_type=jnp.float32, precision=lax.Precision.HIGHEST)
    y4 = linb_ref[...]
    for j in range(16):
        oj = o[:, :, j:j + 1].reshape(_QB, 128)
        y4 = y4 + lax.dot_general(oj, lin_ref[j], (((1,), (1,)), ((), ())),
                                  preferred_element_type=jnp.float32, precision=lax.Precision.HIGHEST)
    y4_ref[...] = y4
    _acc_moments(mom_ref, 128, 0, y4)


def _k5_body(y4_ref, sc_ref, out_ref):
    sc = sc_ref[...]
    out_ref[...] = jnp.maximum(y4_ref[...] * sc[0:1, :] + sc[1:2, :], 0.0)


def _full_spec(shape):
    n = len(shape)
    return pl.BlockSpec(shape, lambda i, _n=n: (0,) * _n)


def _scale_shift(s, sq, cnt, g, be):
    m = s / cnt
    v = sq / cnt - m * m
    scale = g / jnp.sqrt(v + _EPS)
    return scale, be - m * scale


def _pack_sc(pairs, rows=2):
    out = jnp.zeros((rows, 128), jnp.float32)
    for scale, shift, off, r in pairs:
        out = lax.dynamic_update_slice(out, scale[None, :], (r, off))
        out = lax.dynamic_update_slice(out, shift[None, :], (r + 1, off))
    return out


def kernel(xyz, points, params):
    p = params
    inv_d = _density(xyz, jnp.transpose(xyz, (0, 2, 1)))
    fps_idx, new_xyz_cn = _fps(xyz)
    idx_km, gdens_km = _knn(new_xyz_cn, xyz, inv_d)

    idx = jnp.transpose(idx_km, (0, 2, 1))
    flat_idx = (idx + (jnp.arange(_B, dtype=jnp.int32) * _N)[:, None, None]
                ).reshape(_ROWS)
    pts_t = jnp.transpose(points, (0, 2, 1))
    xyz_t = jnp.transpose(xyz, (0, 2, 1))
    table = jnp.concatenate(
        [pts_t, xyz_t, inv_d[:, :, None],
         jnp.zeros((_B, _N, _TBL_C - 68), jnp.float32)], axis=-1
    ).reshape(_B * _N, _TBL_C)

    g = _sc_gather(table, flat_idx)

    nxq = jnp.transpose(new_xyz_cn, (0, 2, 1))
    nxr = jnp.broadcast_to(nxq[:, :, None, :], (_B, _NPOINT, _NSAMPLE, 3)
                           ).reshape(_ROWS, 3)

    cnt = jnp.float32(_ROWS)
    row_spec = pl.BlockSpec((_RB, _TBL_C), lambda i: (i, 0))

    def rs(c):
        return pl.BlockSpec((_RB, c), lambda i: (i, 0))

    mom_spec = pl.BlockSpec((8, 128), lambda i: (0, 0))
    mom_shape = jax.ShapeDtypeStruct((8, 128), jnp.float32)

    y1, wy1, m1 = pl.pallas_call(
        _k1_body,
        grid=(_NBLK,),
        in_specs=[row_spec, rs(3),
                  _full_spec((64, 67)), _full_spec((1, 64)),
                  _full_spec((8, 3)), _full_spec((1, 8))],
        out_specs=[rs(64), rs(8), mom_spec],
        out_shape=[jax.ShapeDtypeStruct((_ROWS, 64), jnp.float32),
                   jax.ShapeDtypeStruct((_ROWS, 8), jnp.float32),
                   mom_shape],
    )(g, nxr, p["mlp0_w"], p["mlp0_b"][None, :], p["wn0_w"],
      p["wn0_b"][None, :])

    s1a, s1b = _scale_shift(m1[0, 0:64], m1[1, 0:64], cnt,
                            p["mlp0_g"], p["mlp0_be"])
    sw1a, sw1b = _scale_shift(m1[0, 64:72], m1[1, 64:72], cnt,
                              p["wn0_g"], p["wn0_be"])
    sc1 = _pack_sc([(s1a, s1b, 0, 0), (sw1a, sw1b, 64, 0)])

    y2, wy2, m2 = pl.pallas_call(
        _k2_body,
        grid=(_NBLK,),
        in_specs=[rs(64), rs(8), _full_spec((2, 128)),
                  _full_spec((64, 64)), _full_spec((1, 64)),
                  _full_spec((8, 8)), _full_spec((1, 8))],
        out_specs=[rs(64), rs(8), mom_spec],
        out_shape=[jax.ShapeDtypeStruct((_ROWS, 64), jnp.float32),
                   jax.ShapeDtypeStruct((_ROWS, 8), jnp.float32),
                   mom_shape],
    )(y1, wy1, sc1, p["mlp1_w"], p["mlp1_b"][None, :], p["wn1_w"],
      p["wn1_b"][None, :])

    s2a, s2b = _scale_shift(m2[0, 0:64], m2[1, 0:64], cnt,
                            p["mlp1_g"], p["mlp1_be"])
    sw2a, sw2b = _scale_shift(m2[0, 64:72], m2[1, 64:72], cnt,
                              p["wn1_g"], p["wn1_be"])
    sc2 = _pack_sc([(s2a, s2b, 0, 0), (sw2a, sw2b, 64, 0)])

    y3, wy3, m3 = pl.pallas_call(
        _k3_body,
        grid=(_NBLK,),
        in_specs=[rs(64), rs(8), _full_spec((2, 128)),
                  _full_spec((128, 64)), _full_spec((1, 128)),
                  _full_spec((16, 8)), _full_spec((1, 16))],
        out_specs=[rs(128), rs(16), mom_spec],
        out_shape=[jax.ShapeDtypeStruct((_ROWS, 128), jnp.float32),
                   jax.ShapeDtypeStruct((_ROWS, 16), jnp.float32),
                   mom_shape],
    )(y2, wy2, sc2, p["mlp2_w"], p["mlp2_b"][None, :], p["wn2_w"],
      p["wn2_b"][None, :])

    s3a, s3b = _scale_shift(m3[0, 0:128], m3[1, 0:128], cnt,
                            p["mlp2_g"], p["mlp2_be"])
    sw3a, sw3b = _scale_shift(m3[2, 0:16], m3[3, 0:16], cnt,
                              p["wn2_g"], p["wn2_be"])
    sc3 = _pack_sc([(s3a, s3b, 0, 0), (sw3a, sw3b, 16, 2)], rows=4)

    dsf = _density_scale(gdens_km, p)

    nq = _B * _NPOINT
    y4, m4 = pl.pallas_call(
        _k4_body,
        grid=(_NBLK,),
        in_specs=[rs(128), rs(1), rs(16), _full_spec((4, 128)),
                  _full_spec((16, 128, 128)), _full_spec((1, 128))],
        out_specs=[pl.BlockSpec((_QB, 128), lambda i: (i, 0)), mom_spec],
        out_shape=[jax.ShapeDtypeStruct((nq, 128), jnp.float32), mom_shape],
    )(y3, dsf, wy3, sc3,
      p["lin_w"].reshape(128, 128, 16).transpose(2, 0, 1),
      p["lin_b"][None, :])

    s4a, s4b = _scale_shift(m4[0, 0:128], m4[1, 0:128], jnp.float32(nq),
                            p["bnl_g"], p["bnl_be"])
    sc4 = _pack_sc([(s4a, s4b, 0, 0)])

    y5 = pl.pallas_call(
        _k5_body,
        grid=(1,),
        in_specs=[_full_spec((nq, 128)), _full_spec((2, 128))],
        out_specs=_full_spec((nq, 128)),
        out_shape=jax.ShapeDtypeStruct((nq, 128), jnp.float32),
    )(y4, sc4)

    x_out = jnp.transpose(y5.reshape(_B, _NPOINT, 128), (0, 2, 1))
    return new_xyz_cn, x_out
